# baseline (speedup 1.0000x reference)
import jax
import jax.numpy as jnp
from jax.experimental import pallas as pl
from jax.experimental.pallas import tpu as pltpu

_NUM_CLASSES = 150
_SMOOTH = 1e-05
_NHI = 16
_NLO = 16


def _hist2d(v_flat, n):
    hi = v_flat >> 4
    lo = v_flat & 15
    hi_iota = jax.lax.broadcasted_iota(jnp.int32, (_NHI, n), 0)
    lo_iota = jax.lax.broadcasted_iota(jnp.int32, (_NLO, n), 0)
    one = jnp.float32(1.0)
    zero = jnp.float32(0.0)
    hi_f = jnp.where(hi == hi_iota, one, zero)
    lo_f = jnp.where(lo == lo_iota, one, zero)
    return jax.lax.dot_general(
        hi_f, lo_f, (((1,), (1,)), ((), ())),
        preferred_element_type=jnp.float32,
    )


def _iou_kernel(xa_ref, xb_ref, t_ref, out_ref, acc_ref):
    b = pl.program_id(0)
    i = pl.program_id(1)
    nb = pl.num_programs(0)
    ni = pl.num_programs(1)

    @pl.when((b == 0) & (i == 0))
    def _init():
        acc_ref[...] = jnp.zeros_like(acc_ref)

    xa = xa_ref[0]
    xb = xb_ref[0]
    t = t_ref[0]
    ch, r, w = xa.shape
    c = 2 * ch
    n = r * w

    runmax = xa[0]
    runidx = jnp.zeros((r, w), jnp.int32)
    for ci in range(1, c):
        xi = xa[ci] if ci < ch else xb[ci - ch]
        gt = xi > runmax
        runmax = jnp.maximum(runmax, xi)
        runidx = jnp.where(gt, ci, runidx)
    pred = runidx

    t_flat = t.reshape(1, n)
    p_flat = pred.reshape(1, n)
    eq = t_flat == p_flat
    t_masked = jnp.where(eq, t_flat, c)

    acc_ref[0] += _hist2d(t_masked, n)
    acc_ref[1] += _hist2d(p_flat, n)
    acc_ref[2] += _hist2d(t_flat, n)

    @pl.when((b == nb - 1) & (i == ni - 1))
    def _fin():
        inter = acc_ref[0].reshape(1, _NHI * _NLO)
        cnt_p = acc_ref[1].reshape(1, _NHI * _NLO)
        cnt_t = acc_ref[2].reshape(1, _NHI * _NLO)
        out_ref[...] = (inter / (cnt_p + cnt_t - inter + _SMOOTH))[0, :_NUM_CLASSES]


def kernel(pred_logits, target):
    B, C, H, W = pred_logits.shape
    R = 64
    nblk = H // R
    out = pl.pallas_call(
        _iou_kernel,
        grid=(B, nblk),
        in_specs=[
            pl.BlockSpec((1, C // 2, R, W), lambda b, i: (b, 0, i, 0)),
            pl.BlockSpec((1, C // 2, R, W), lambda b, i: (b, 1, i, 0)),
            pl.BlockSpec((1, R, W), lambda b, i: (b, i, 0)),
        ],
        out_specs=pl.BlockSpec((C,), lambda b, i: (0,)),
        out_shape=jax.ShapeDtypeStruct((C,), jnp.float32),
        scratch_shapes=[pltpu.VMEM((3, _NHI, _NLO), jnp.float32)],
    )(pred_logits, pred_logits, target)
    return out


# speedup vs baseline: 1.0209x; 1.0209x over previous
import jax
import jax.numpy as jnp
from jax.experimental import pallas as pl
from jax.experimental.pallas import tpu as pltpu

_NUM_CLASSES = 150
_SMOOTH = 1e-05
_NHI = 16
_NLO = 16


def _hist2d(v_flat, n):
    hi = v_flat >> 4
    lo = v_flat & 15
    hi_iota = jax.lax.broadcasted_iota(jnp.int32, (_NHI, n), 0)
    lo_iota = jax.lax.broadcasted_iota(jnp.int32, (_NLO, n), 0)
    one = jnp.float32(1.0)
    zero = jnp.float32(0.0)
    hi_f = jnp.where(hi == hi_iota, one, zero)
    lo_f = jnp.where(lo == lo_iota, one, zero)
    return jax.lax.dot_general(
        hi_f, lo_f, (((1,), (1,)), ((), ())),
        preferred_element_type=jnp.float32,
    )


def _iou_kernel(x_ref, t_ref, out_ref, acc_ref):
    b = pl.program_id(0)
    i = pl.program_id(1)
    nb = pl.num_programs(0)
    ni = pl.num_programs(1)

    @pl.when((b == 0) & (i == 0))
    def _init():
        acc_ref[...] = jnp.zeros_like(acc_ref)

    x = x_ref[0]
    t = t_ref[0]
    c, r, w = x.shape
    n = r * w

    runmax = x[0]
    runidx = jnp.zeros((r, w), jnp.int32)
    for ci in range(1, c):
        xi = x[ci]
        gt = xi > runmax
        runmax = jnp.maximum(runmax, xi)
        runidx = jnp.where(gt, ci, runidx)
    pred = runidx

    t_flat = t.reshape(1, n)
    p_flat = pred.reshape(1, n)
    eq = t_flat == p_flat
    t_masked = jnp.where(eq, t_flat, c)

    acc_ref[0] += _hist2d(t_masked, n)
    acc_ref[1] += _hist2d(p_flat, n)
    acc_ref[2] += _hist2d(t_flat, n)

    @pl.when((b == nb - 1) & (i == ni - 1))
    def _fin():
        inter = acc_ref[0].reshape(1, _NHI * _NLO)
        cnt_p = acc_ref[1].reshape(1, _NHI * _NLO)
        cnt_t = acc_ref[2].reshape(1, _NHI * _NLO)
        out_ref[...] = (inter / (cnt_p + cnt_t - inter + _SMOOTH))[0, :_NUM_CLASSES]


def kernel(pred_logits, target):
    B, C, H, W = pred_logits.shape
    R = 64
    nblk = H // R
    out = pl.pallas_call(
        _iou_kernel,
        grid=(B, nblk),
        in_specs=[
            pl.BlockSpec((1, C, R, W), lambda b, i: (b, 0, i, 0)),
            pl.BlockSpec((1, R, W), lambda b, i: (b, i, 0)),
        ],
        out_specs=pl.BlockSpec((C,), lambda b, i: (0,)),
        out_shape=jax.ShapeDtypeStruct((C,), jnp.float32),
        scratch_shapes=[pltpu.VMEM((3, _NHI, _NLO), jnp.float32)],
    )(pred_logits, target)
    return out
